# Initial kernel scaffold; baseline (speedup 1.0000x reference)
import jax
import jax.numpy as jnp
from jax.experimental import pallas as pl


def kernel(nodes, edges, senders, receivers, W1, b1, W2, b2, W3, b3):
    raise NotImplementedError("write your pallas kernel here")


# speedup vs baseline: 2.6717x; 2.6717x over previous
import functools

import jax
import jax.numpy as jnp
from jax import lax
from jax.experimental import pallas as pl
from jax.experimental.pallas import tpu as pltpu
from jax.experimental.pallas import tpu_sc as plsc

N = 10000
E = 320000
DN = 128
DE = 16
H1 = 256
NPAD = 10240
NC, NS = 2, 16
NW = NC * NS
EW = E // NW
CH = 80
RPT = NPAD // NS
WBC = 8


def _mesh():
    return plsc.VectorSubcoreMesh(core_axis_name="c", subcore_axis_name="s",
                                  num_cores=NC, num_subcores=NS)


@functools.cache
def _sc_gather_kernel():
    return pl.kernel(
        _sc_gather_body,
        out_type=(
            jax.ShapeDtypeStruct((E, DN), jnp.float32),
            jax.ShapeDtypeStruct((E, DN), jnp.float32),
            jax.ShapeDtypeStruct((NC, NPAD, DN), jnp.float32),
        ),
        mesh=_mesh(),
        scratch_types=[
            pltpu.VMEM((CH,), jnp.int32),
            pltpu.VMEM((CH,), jnp.int32),
            pltpu.VMEM((CH, DN), jnp.float32),
            pltpu.VMEM((CH, DN), jnp.float32),
            pltpu.VMEM((CH, DN), jnp.float32),
            pltpu.VMEM((RPT // WBC, DN), jnp.float32),
            pltpu.VMEM_SHARED((NPAD, DN), jnp.float32),
            pltpu.SemaphoreType.DMA,
            pltpu.SemaphoreType.DMA,
        ],
    )


def _sc_gather_body(nodes_hbm, r_hbm, s_hbm, zeros_hbm, ones_hbm,
                    a_out, c_out, cnt_out,
                    r_idx, s_idx, a_buf, c_buf, ones_buf, wb_buf, acc,
                    sem_a, sem_c):
    cid = lax.axis_index("c")
    sid = lax.axis_index("s")
    wid = cid * NS + sid
    base = wid * EW
    tb = sid * RPT

    pltpu.sync_copy(zeros_hbm.at[pl.ds(tb, RPT)], acc.at[pl.ds(tb, RPT)])
    pltpu.sync_copy(ones_hbm, ones_buf)
    plsc.subcore_barrier()

    def body(j, carry):
        cbase = base + j * CH
        pltpu.sync_copy(r_hbm.at[pl.ds(cbase, CH)], r_idx)
        pltpu.sync_copy(s_hbm.at[pl.ds(cbase, CH)], s_idx)
        ca = pltpu.async_copy(nodes_hbm.at[r_idx], a_buf, sem_a)
        cc = pltpu.async_copy(nodes_hbm.at[s_idx], c_buf, sem_c)
        pltpu.sync_copy(ones_buf, acc.at[r_idx], add=True)
        ca.wait()
        cc.wait()
        pltpu.sync_copy(a_buf, a_out.at[pl.ds(cbase, CH)])
        pltpu.sync_copy(c_buf, c_out.at[pl.ds(cbase, CH)])
        return carry

    lax.fori_loop(0, EW // CH, body, 0)
    plsc.subcore_barrier()

    def wb(k, carry):
        r0 = tb + k * (RPT // WBC)
        pltpu.sync_copy(acc.at[pl.ds(r0, RPT // WBC)], wb_buf)
        pltpu.sync_copy(wb_buf, cnt_out.at[cid, pl.ds(r0, RPT // WBC)])
        return carry

    lax.fori_loop(0, WBC, wb, 0)


@functools.cache
def _sc_scatter_kernel():
    return pl.kernel(
        _sc_scatter_body,
        out_type=jax.ShapeDtypeStruct((NC, NPAD, DN), jnp.float32),
        mesh=_mesh(),
        scratch_types=[
            pltpu.VMEM((CH,), jnp.int32),
            pltpu.VMEM((CH, DN), jnp.float32),
            pltpu.VMEM((RPT // WBC, DN), jnp.float32),
            pltpu.VMEM_SHARED((NPAD, DN), jnp.float32),
        ],
    )


def _sc_scatter_body(e2_hbm, r_hbm, zeros_hbm, p_out,
                     r_idx, row_buf, wb_buf, acc):
    cid = lax.axis_index("c")
    sid = lax.axis_index("s")
    wid = cid * NS + sid
    base = wid * EW
    tb = sid * RPT

    pltpu.sync_copy(zeros_hbm.at[pl.ds(tb, RPT)], acc.at[pl.ds(tb, RPT)])
    plsc.subcore_barrier()

    def body(j, carry):
        cbase = base + j * CH
        pltpu.sync_copy(r_hbm.at[pl.ds(cbase, CH)], r_idx)
        pltpu.sync_copy(e2_hbm.at[pl.ds(cbase, CH)], row_buf)
        pltpu.sync_copy(row_buf, acc.at[r_idx], add=True)
        return carry

    lax.fori_loop(0, EW // CH, body, 0)
    plsc.subcore_barrier()

    def wb(k, carry):
        r0 = tb + k * (RPT // WBC)
        pltpu.sync_copy(acc.at[pl.ds(r0, RPT // WBC)], wb_buf)
        pltpu.sync_copy(wb_buf, p_out.at[cid, pl.ds(r0, RPT // WBC)])
        return carry

    lax.fori_loop(0, WBC, wb, 0)


def _mlp_body(a_ref, c_ref, e_ref, w1_ref, b1_ref, w2_ref, b2_ref,
              w3_ref, b3_ref, e2_ref, eo_ref):
    w1 = w1_ref[...]
    h = jnp.dot(a_ref[...], w1[0:DN, :], preferred_element_type=jnp.float32)
    h += jnp.dot(e_ref[...], w1[DN:DN + DE, :], preferred_element_type=jnp.float32)
    h += jnp.dot(c_ref[...], w1[DN + DE:, :], preferred_element_type=jnp.float32)
    h = jax.nn.relu(h + b1_ref[...])
    e2 = jax.nn.relu(jnp.dot(h, w2_ref[...], preferred_element_type=jnp.float32)
                     + b2_ref[...])
    e2_ref[...] = e2
    eo_ref[...] = jax.nn.relu(
        jnp.dot(e2, w3_ref[...], preferred_element_type=jnp.float32) + b3_ref[...])


def _tc_mlp(a, c, e, w1, b1, w2, b2, w3, b3, te=1280):
    grid = E // te
    blk = lambda d: pl.BlockSpec((te, d), lambda i: (i, 0))
    full = lambda s: pl.BlockSpec(s, lambda i: (0,) * len(s))
    return pl.pallas_call(
        _mlp_body,
        grid=(grid,),
        in_specs=[
            blk(DN), blk(DN), blk(DE),
            full((DN + DE + DN, H1)), full((1, H1)),
            full((H1, DN)), full((1, DN)),
            full((DN, DE)), full((1, DE)),
        ],
        out_specs=[blk(DN), blk(DE)],
        out_shape=[
            jax.ShapeDtypeStruct((E, DN), jnp.float32),
            jax.ShapeDtypeStruct((E, DE), jnp.float32),
        ],
    )(a, c, e, w1, b1, w2, b2, w3, b3)


def _combine_body(p_ref, cnt_ref, o_ref):
    s = p_ref[0, 0:N, :] + p_ref[1, 0:N, :]
    cnt = cnt_ref[0, 0:N, 0:1] + cnt_ref[1, 0:N, 0:1]
    o_ref[...] = s / jnp.maximum(cnt, 1.0)


def _tc_combine(p, cnt):
    return pl.pallas_call(
        _combine_body,
        out_shape=jax.ShapeDtypeStruct((N, DN), jnp.float32),
    )(p, cnt)


def kernel(nodes, edges, senders, receivers, W1, b1, W2, b2, W3, b3):
    b = nodes.shape[0]
    nodes_flat = nodes.reshape(N, DN)
    edges_flat = edges.reshape(E, DE)
    r = receivers.reshape(E)
    s = senders.reshape(E)

    zeros = jnp.zeros((NPAD, DN), jnp.float32)
    ones = jnp.ones((CH, DN), jnp.float32)
    a_gath, c_gath, cnt = _sc_gather_kernel()(nodes_flat, r, s, zeros, ones)
    e2, edges_out = _tc_mlp(
        a_gath, c_gath, edges_flat, W1, b1.reshape(1, H1),
        W2, b2.reshape(1, DN), W3, b3.reshape(1, DE))
    p = _sc_scatter_kernel()(e2, r, zeros)
    nodes_out = _tc_combine(p, cnt)
    return (nodes_out.reshape(b, N, DN), edges_out.reshape(b, E, DE),
            senders, receivers)
